# baseline (speedup 1.0000x reference)
import functools

import jax
import jax.numpy as jnp
from jax import lax
from jax.experimental import pallas as pl
from jax.experimental.pallas import tpu as pltpu
from jax.experimental.pallas import tpu_sc as plsc

VOCAB_DIM = 8192
TOKENS = 4 * 2048
NC, NS = 2, 16
NW = NC * NS
TPW = TOKENS // NW
L = 16
GROUPS = TPW // L


def _body(idx_hbm, table_hbm, out_hbm, idx_v, *sems):
    wid = lax.axis_index("s") * NC + lax.axis_index("c")
    pltpu.sync_copy(idx_hbm.at[pl.ds(wid * TPW, TPW)], idx_v)
    base = wid * TPW
    lanes = lax.iota(jnp.int32, 16)

    def step(g, carry):
        vec = idx_v[pl.ds(g * L, L)]
        for l in range(L):
            row = vec[l]
            j = g * L + l

            @pl.when(g >= 1)
            def _drain():
                pltpu.make_async_copy(
                    table_hbm.at[pl.ds(0, 1)],
                    out_hbm.at[pl.ds(base, 1)], sems[l]).wait()

            pltpu.async_copy(
                table_hbm.at[pl.ds(row, 1)],
                out_hbm.at[pl.ds(base + j, 1)], sems[l])
        return carry

    lax.fori_loop(0, GROUPS, step, 0, unroll=False)
    for l in range(L):
        pltpu.make_async_copy(
            table_hbm.at[pl.ds(0, 1)],
            out_hbm.at[pl.ds(base, 1)], sems[l]).wait()


_mesh = plsc.VectorSubcoreMesh(core_axis_name="c", subcore_axis_name="s")

_gather = functools.partial(
    pl.kernel,
    mesh=_mesh,
    out_type=jax.ShapeDtypeStruct((TOKENS, VOCAB_DIM), jnp.float32),
    scratch_types=(
        [pltpu.VMEM((TPW,), jnp.int32)]
        + [pltpu.SemaphoreType.DMA] * L
    ),
)(_body)


def kernel(x, vocab_table):
    idx = x.reshape(TOKENS).astype(jnp.int32)
    out = _gather(idx, vocab_table)
    return out.reshape(x.shape + (VOCAB_DIM,))


# speedup vs baseline: 39.5792x; 39.5792x over previous
import functools

import jax
import jax.numpy as jnp
from jax import lax
from jax.experimental import pallas as pl
from jax.experimental.pallas import tpu as pltpu
from jax.experimental.pallas import tpu_sc as plsc

VOCAB_DIM = 8192
TOKENS = 4 * 2048
NC, NS = 2, 16
NW = NC * NS
TPW = TOKENS // NW
K = 4
CHUNKS = TPW // K


def _body(idx_hbm, table_hbm, out_hbm, idx_v,
          buf0, buf1, buf2, sg0, sg1, sg2, sw0, sw1, sw2):
    wid = lax.axis_index("s") * NC + lax.axis_index("c")
    pltpu.sync_copy(idx_hbm.at[pl.ds(wid * CHUNKS, CHUNKS)], idx_v)
    base = wid * TPW
    bufs = (buf0, buf1, buf2)
    sgs = (sg0, sg1, sg2)
    sws = (sw0, sw1, sw2)

    pltpu.async_copy(table_hbm.at[idx_v.at[0]], buf0, sg0)
    pltpu.async_copy(table_hbm.at[idx_v.at[1]], buf1, sg1)

    def step(g, carry):
        for b in range(3):
            j = 3 * g + b
            p, pn = b, (b + 2) % 3
            @pl.when(j >= 1)
            def _drain():
                pltpu.make_async_copy(
                    bufs[pn], out_hbm.at[pl.ds(base, K)], sws[pn]).wait()

            @pl.when(j + 2 < CHUNKS)
            def _prefetch():
                pltpu.async_copy(
                    table_hbm.at[idx_v.at[j + 2]], bufs[pn], sgs[pn])

            pltpu.make_async_copy(
                table_hbm.at[idx_v.at[j]], bufs[p], sgs[p]).wait()
            pltpu.async_copy(bufs[p], out_hbm.at[pl.ds(base + j * K, K)],
                             sws[p])
        return carry

    lax.fori_loop(0, CHUNKS // 3, step, 0, unroll=False)
    j = CHUNKS - 1
    pltpu.make_async_copy(table_hbm.at[idx_v.at[j]], bufs[0], sgs[0]).wait()
    pltpu.async_copy(bufs[0], out_hbm.at[pl.ds(base + j * K, K)], sws[0])
    pltpu.make_async_copy(bufs[2], out_hbm.at[pl.ds(base, K)], sws[2]).wait()
    pltpu.make_async_copy(bufs[0], out_hbm.at[pl.ds(base, K)], sws[0]).wait()


_mesh = plsc.VectorSubcoreMesh(core_axis_name="c", subcore_axis_name="s")

_gather = functools.partial(
    pl.kernel,
    mesh=_mesh,
    out_type=jax.ShapeDtypeStruct((TOKENS, VOCAB_DIM), jnp.float32),
    scratch_types=[
        pltpu.VMEM((CHUNKS, K), jnp.int32),
        pltpu.VMEM((K, VOCAB_DIM), jnp.float32),
        pltpu.VMEM((K, VOCAB_DIM), jnp.float32),
        pltpu.VMEM((K, VOCAB_DIM), jnp.float32),
        pltpu.SemaphoreType.DMA,
        pltpu.SemaphoreType.DMA,
        pltpu.SemaphoreType.DMA,
        pltpu.SemaphoreType.DMA,
        pltpu.SemaphoreType.DMA,
        pltpu.SemaphoreType.DMA,
    ],
)(_body)


def kernel(x, vocab_table):
    idx = x.reshape(TOKENS).astype(jnp.int32).reshape(NW * CHUNKS, K)
    out = _gather(idx, vocab_table)
    return out.reshape(x.shape + (VOCAB_DIM,))
